# Initial kernel scaffold; baseline (speedup 1.0000x reference)
import jax
import jax.numpy as jnp
from jax.experimental import pallas as pl


def kernel(nh, eh, edge_index, W1, b1, W2, b2):
    raise NotImplementedError("write your pallas kernel here")


# speedup vs baseline: 5.2027x; 5.2027x over previous
import functools

import jax
import jax.numpy as jnp
from jax import lax
from jax.experimental import pallas as pl
from jax.experimental.pallas import tpu as pltpu
from jax.experimental.pallas import tpu_sc as plsc

N = 10000
E = 320000
D = 128

NC = 2
NS = 16
NW = NC * NS
EW = E // NW
CH = 80
NCH = EW // CH
NH = N // NC
ET = E // NS
NCH_T = ET // CH
ACC_ROWS = NH + CH
RPT = 312
RPT_LAST = NH - (NS - 1) * RPT

_sc_mesh = plsc.VectorSubcoreMesh(core_axis_name="c", subcore_axis_name="s")
_sc_params = pltpu.CompilerParams(needs_layout_passes=False)


@functools.partial(
    pl.kernel,
    out_type=jax.ShapeDtypeStruct((NW * N,), jnp.float32),
    mesh=_sc_mesh,
    compiler_params=_sc_params,
    scratch_types=[
        pltpu.VMEM((CH,), jnp.int32),
        pltpu.VMEM((N,), jnp.float32),
    ],
)
def _deg_kernel(dst_hbm, out_hbm, idx_v, bins_v):
    cid = lax.axis_index("c")
    sid = lax.axis_index("s")
    wid = cid * NS + sid
    zero16 = jnp.zeros((16,), jnp.float32)
    one16 = jnp.ones((16,), jnp.float32)

    @pl.loop(0, N // 16)
    def _zero(i):
        bins_v[pl.ds(i * 16, 16)] = zero16

    @pl.loop(0, NCH)
    def _chunks(i):
        base = wid * EW + i * CH
        pltpu.sync_copy(dst_hbm.at[pl.ds(base, CH)], idx_v)
        for j in range(CH // 16):
            idx16 = idx_v[pl.ds(j * 16, 16)]
            plsc.addupdate_scatter(bins_v, [idx16], one16)

    pltpu.sync_copy(bins_v, out_hbm.at[pl.ds(wid * N, N)])


def _dis_body(degp_ref, out_ref):
    deg = jnp.sum(degp_ref[...], axis=0)
    dis = jnp.where(deg > 0, lax.rsqrt(jnp.maximum(deg, 1e-12)), 0.0)
    out_ref[...] = jnp.broadcast_to(dis[:, None], (N, D))


def _dis_bcast(degp):
    return pl.pallas_call(
        _dis_body,
        out_shape=jax.ShapeDtypeStruct((N, D), jnp.float32),
    )(degp)


_RB = 1000


def _scale_body(nh_ref, dis_ref, out_ref):
    out_ref[...] = nh_ref[...] * dis_ref[...]


def _scale(nh, dis_b):
    return pl.pallas_call(
        _scale_body,
        grid=(N // _RB,),
        in_specs=[
            pl.BlockSpec((_RB, D), lambda i: (i, 0)),
            pl.BlockSpec((_RB, D), lambda i: (i, 0)),
        ],
        out_specs=pl.BlockSpec((_RB, D), lambda i: (i, 0)),
        out_shape=jax.ShapeDtypeStruct((N, D), jnp.float32),
    )(nh, dis_b)


@functools.partial(
    pl.kernel,
    out_type=jax.ShapeDtypeStruct((N, D), jnp.float32),
    mesh=_sc_mesh,
    compiler_params=_sc_params,
    scratch_types=[
        pltpu.VMEM((CH,), jnp.int32),
        pltpu.VMEM((CH,), jnp.int32),
        pltpu.VMEM((CH,), jnp.int32),
        pltpu.VMEM((CH, D), jnp.float32),
        pltpu.VMEM((RPT_LAST, D), jnp.float32),
        pltpu.VMEM_SHARED((ACC_ROWS, D), jnp.float32),
        pltpu.SemaphoreType.DMA,
    ],
)
def _agg_kernel(nhs_hbm, src_hbm, dst_hbm, out_hbm,
                idxs_v, idxd_v, idxl_v, rows_v, zbuf_v, acc_sh, sem):
    cid = lax.axis_index("c")
    sid = lax.axis_index("s")
    zero16 = jnp.zeros((16,), jnp.float32)
    iota16 = lax.iota(jnp.int32, 16)
    lo = cid * NH

    @pl.loop(0, RPT_LAST)
    def _zero(r):
        for j in range(D // 16):
            zbuf_v[r, pl.ds(j * 16, 16)] = zero16

    @pl.when(sid < NS - 1)
    def _():
        pltpu.sync_copy(zbuf_v.at[pl.ds(0, RPT)], acc_sh.at[pl.ds(sid * RPT, RPT)])

    @pl.when(sid == NS - 1)
    def _():
        pltpu.sync_copy(zbuf_v, acc_sh.at[pl.ds((NS - 1) * RPT, RPT_LAST)])
        pltpu.sync_copy(zbuf_v.at[pl.ds(0, CH)], acc_sh.at[pl.ds(NH, CH)])

    plsc.subcore_barrier()

    @pl.loop(0, NCH_T)
    def _chunks(i):
        base = sid * ET + i * CH
        pltpu.sync_copy(src_hbm.at[pl.ds(base, CH)], idxs_v)
        pltpu.sync_copy(dst_hbm.at[pl.ds(base, CH)], idxd_v)
        for j in range(CH // 16):
            d16 = idxd_v[pl.ds(j * 16, 16)]
            loc = d16 - lo
            in_range = (d16 >= lo) & (loc < NH)
            idxl_v[pl.ds(j * 16, 16)] = jnp.where(
                in_range, loc, NH + j * 16 + iota16)
        pltpu.async_copy(nhs_hbm.at[idxs_v], rows_v, sem).wait()
        pltpu.sync_copy(rows_v, acc_sh.at[idxl_v], add=True)

    plsc.subcore_barrier()

    @pl.when(sid < NS - 1)
    def _():
        pltpu.sync_copy(acc_sh.at[pl.ds(sid * RPT, RPT)],
                        out_hbm.at[pl.ds(lo + sid * RPT, RPT)])

    @pl.when(sid == NS - 1)
    def _():
        pltpu.sync_copy(acc_sh.at[pl.ds((NS - 1) * RPT, RPT_LAST)],
                        out_hbm.at[pl.ds(lo + (NS - 1) * RPT, RPT_LAST)])


def _mlp_body(agg_ref, dis_ref, W1_ref, b1_ref, W2_ref, b2_ref, out_ref):
    nh_ = agg_ref[...] * dis_ref[...]
    h = jnp.maximum(
        jnp.dot(nh_, W1_ref[...], preferred_element_type=jnp.float32)
        + b1_ref[...], 0.0)
    out_ref[...] = (
        jnp.dot(h, W2_ref[...], preferred_element_type=jnp.float32)
        + b2_ref[...])


def _mlp(agg, dis_b, W1, b1, W2, b2):
    return pl.pallas_call(
        _mlp_body,
        grid=(N // _RB,),
        in_specs=[
            pl.BlockSpec((_RB, D), lambda i: (i, 0)),
            pl.BlockSpec((_RB, D), lambda i: (i, 0)),
            pl.BlockSpec((D, D), lambda i: (0, 0)),
            pl.BlockSpec((1, D), lambda i: (0, 0)),
            pl.BlockSpec((D, D), lambda i: (0, 0)),
            pl.BlockSpec((1, D), lambda i: (0, 0)),
        ],
        out_specs=pl.BlockSpec((_RB, D), lambda i: (i, 0)),
        out_shape=jax.ShapeDtypeStruct((N, D), jnp.float32),
    )(agg, dis_b, W1, b1, W2, b2)


def kernel(nh, eh, edge_index, W1, b1, W2, b2):
    src = edge_index[0]
    dst = edge_index[1]
    degp = _deg_kernel(dst).reshape(NW, N)
    dis_b = _dis_bcast(degp)
    nhs = _scale(nh, dis_b)
    agg = _agg_kernel(nhs, src, dst)
    out = _mlp(agg, dis_b, W1, b1.reshape(1, D), W2, b2.reshape(1, D))
    return (out, eh)
